# Initial kernel scaffold; baseline (speedup 1.0000x reference)
import jax
import jax.numpy as jnp
from jax.experimental import pallas as pl


def kernel(node_ftr, edge_ftr, edge_data, W_ff_node, b_ff_node, W_ff_edge, b_ff_edge, W_ff_gbl, b_ff_gbl, We, be, Wv, bv, Wu, bu, Wo, bo, atom4bond, bond4atom, combinations):
    raise NotImplementedError("write your pallas kernel here")


# speedup vs baseline: 2.5458x; 2.5458x over previous
import functools

import jax
import jax.numpy as jnp
from jax import lax
from jax.experimental import pallas as pl
from jax.experimental.pallas import tpu as pltpu
from jax.experimental.pallas import tpu_sc as plsc

N = 10000
E = 320000
H = 64
C = 1024
P = 16
CP = C * P

NUM_CORES = 2
NUM_SUBCORES = 16
NW = NUM_CORES * NUM_SUBCORES
EB = 128
NBATCH = E // EB
NB_BASE = NBATCH // NW
NB_REM = NBATCH % NW
N_PAD = 10112
ROWS_PER_TILE = N_PAD // NUM_SUBCORES

_F32 = jnp.float32


def _tc_node_body(nf, wn, bn, wes, wed, weg, be_, wv2, bv_, bg,
                  o_n0, o_ab, o_eb, o_vb):
    n0 = jnp.maximum(
        jnp.dot(nf[...], wn[...], preferred_element_type=_F32, precision=lax.Precision.HIGHEST) + bn[...], 0.0)
    o_n0[...] = n0
    a = jnp.dot(n0, wes[...], preferred_element_type=_F32, precision=lax.Precision.HIGHEST)
    b = jnp.dot(n0, wed[...], preferred_element_type=_F32, precision=lax.Precision.HIGHEST)
    o_ab[...] = jnp.concatenate([a, b], axis=1)
    g = jnp.maximum(bg[...], 0.0)
    o_eb[...] = jnp.dot(g, weg[...], preferred_element_type=_F32, precision=lax.Precision.HIGHEST) + be_[...]
    o_vb[...] = jnp.dot(g, wv2[...], preferred_element_type=_F32, precision=lax.Precision.HIGHEST) + bv_[...]


def _tc_edge_body(ef, we1, be1, wee, eb, o):
    e0 = jnp.maximum(
        jnp.dot(ef[...], we1[...], preferred_element_type=_F32, precision=lax.Precision.HIGHEST) + be1[...], 0.0)
    o[...] = jnp.dot(e0, wee[...], preferred_element_type=_F32, precision=lax.Precision.HIGHEST) + eb[...]


def _tc2_body(ag0, ag1, n0r, wv0, wv1, vb, w2, ed, woe, bo_, o_s2, o_t):
    aggsum = ag0[...] + ag1[...]
    nn = jnp.maximum(
        jnp.dot(aggsum, wv0[...], preferred_element_type=_F32, precision=lax.Precision.HIGHEST)
        + jnp.dot(n0r[...], wv1[...], preferred_element_type=_F32, precision=lax.Precision.HIGHEST)
        + vb[...], 0.0)
    nf = n0r[...] + nn
    o_s2[...] = jnp.dot(nf, w2[...], preferred_element_type=_F32, precision=lax.Precision.HIGHEST)
    o_t[...] = jnp.dot(ed[...], woe[...], preferred_element_type=_F32, precision=lax.Precision.HIGHEST) + bo_[...]


_CP_PER_W = CP // NW


@functools.cache
def _sc_kernels():
    mesh = plsc.VectorSubcoreMesh(
        core_axis_name="c", subcore_axis_name="s",
        num_cores=NUM_CORES, num_subcores=NUM_SUBCORES)

    @functools.partial(
        pl.kernel,
        mesh=mesh,
        out_type=jax.ShapeDtypeStruct((NUM_CORES, N_PAD, 2 * H), _F32),
        scratch_types=[
            pltpu.VMEM((1, EB), jnp.int32),
            pltpu.VMEM((1, EB), jnp.int32),
            pltpu.VMEM((1, EB), jnp.int32),
            pltpu.VMEM((EB, 2 * H), _F32),
            pltpu.VMEM((EB, 2 * H), _F32),
            pltpu.VMEM((EB, H), _F32),
            pltpu.VMEM_SHARED((N_PAD, 2 * H), _F32),
            pltpu.SemaphoreType.DMA,
            pltpu.SemaphoreType.DMA,
        ],
    )
    def sc_edge_kernel(t_hbm, ep_hbm, src_hbm, dst_hbm, seg_hbm,
                       zero_hbm, out_hbm, idxs, idxd, idxg, ra, rb, re,
                       agg_sh, sem1, sem2):
        cid = lax.axis_index("c")
        sid = lax.axis_index("s")
        wid = sid * NUM_CORES + cid
        r0 = sid * ROWS_PER_TILE
        pltpu.sync_copy(zero_hbm.at[pl.ds(r0, ROWS_PER_TILE)],
                        agg_sh.at[pl.ds(r0, ROWS_PER_TILE)])
        plsc.subcore_barrier()

        nb = NB_BASE + jnp.where(wid < NB_REM, 1, 0)

        def batch_body(j, carry):
            base = (wid + NW * j) * EB
            pltpu.sync_copy(src_hbm.at[pl.ds(base, EB)], idxs.at[0])
            pltpu.sync_copy(dst_hbm.at[pl.ds(base, EB)], idxd.at[0])
            pltpu.sync_copy(seg_hbm.at[pl.ds(base, EB)], idxg.at[0])
            cp_a = pltpu.async_copy(t_hbm.at[idxs.at[0]], ra, sem1)
            cp_b = pltpu.async_copy(t_hbm.at[idxd.at[0]], rb, sem2)
            pltpu.sync_copy(ep_hbm.at[pl.ds(base, EB)], re)
            cp_a.wait()
            cp_b.wait()

            def row_body(r, c2):
                for q in range(H // 16):
                    sl = pl.ds(q * 16, 16)
                    slb = pl.ds(H + q * 16, 16)
                    ra[r, sl] = jnp.maximum(
                        ra[r, sl] + rb[r, slb] + re[r, sl], 0.0)
                return c2

            lax.fori_loop(0, EB, row_body, 0)
            pltpu.sync_copy(ra, agg_sh.at[idxg.at[0]], add=True)
            return carry

        lax.fori_loop(0, nb, batch_body, 0)
        plsc.subcore_barrier()
        pltpu.sync_copy(agg_sh.at[pl.ds(r0, ROWS_PER_TILE)],
                        out_hbm.at[cid, pl.ds(r0, ROWS_PER_TILE)])

    @functools.partial(
        pl.kernel,
        mesh=mesh,
        out_type=jax.ShapeDtypeStruct((CP,), _F32),
        compiler_params=pltpu.CompilerParams(needs_layout_passes=False),
        scratch_types=[
            pltpu.VMEM((2 * N,), _F32),
            pltpu.VMEM((_CP_PER_W,), jnp.int32),
            pltpu.VMEM((_CP_PER_W,), jnp.int32),
            pltpu.VMEM((_CP_PER_W,), _F32),
            pltpu.VMEM((_CP_PER_W,), _F32),
        ],
    )
    def sc_readout_kernel(s2_hbm, t_hbm, i0_hbm, i1_hbm, out_hbm,
                          s2v, i0v, i1v, tv, ov):
        cid = lax.axis_index("c")
        sid = lax.axis_index("s")
        wid = sid * NUM_CORES + cid
        base = wid * _CP_PER_W
        pltpu.sync_copy(s2_hbm, s2v)
        pltpu.sync_copy(i0_hbm.at[pl.ds(base, _CP_PER_W)], i0v)
        pltpu.sync_copy(i1_hbm.at[pl.ds(base, _CP_PER_W)], i1v)
        pltpu.sync_copy(t_hbm.at[pl.ds(base, _CP_PER_W)], tv)

        def body(i, c):
            sl = pl.ds(i * 16, 16)
            a0 = i0v[sl] * 2
            a1 = i1v[sl] * 2 + 1
            v0 = plsc.load_gather(s2v, [a0])
            v1 = plsc.load_gather(s2v, [a1])
            ov[sl] = v0 + v1 + tv[sl]
            return c

        lax.fori_loop(0, _CP_PER_W // 16, body, 0)
        pltpu.sync_copy(ov, out_hbm.at[pl.ds(base, _CP_PER_W)])

    return sc_edge_kernel, sc_readout_kernel


def kernel(node_ftr, edge_ftr, edge_data, W_ff_node, b_ff_node, W_ff_edge,
           b_ff_edge, W_ff_gbl, b_ff_gbl, We, be, Wv, bv, Wu, bu, Wo, bo,
           atom4bond, bond4atom, combinations):
    nf = node_ftr[0]
    ef = edge_ftr[0]
    src = atom4bond[0]
    dst = atom4bond[1]
    i0 = combinations[..., 0].reshape(-1)
    i1 = combinations[..., 1].reshape(-1)
    ed = edge_data.reshape(CP, 4)

    We_s, We_d, We_e, We_g = We[0:H], We[H:2 * H], We[2 * H:3 * H], We[3 * H:]
    Wv0, Wv1, Wv2 = Wv[0:H], Wv[H:2 * H], Wv[2 * H:]
    W2 = jnp.stack([Wo[0:H, 0], Wo[H:2 * H, 0]], axis=1)
    Woe = Wo[2 * H:, :]

    bn = b_ff_node[None, :]
    be1 = b_ff_edge[None, :]
    be_ = be[None, :]
    bv_ = bv[None, :]
    bg = b_ff_gbl[None, :]
    bo_ = bo[None, :]

    shp = jax.ShapeDtypeStruct
    node0, T, ebias, vbias = pl.pallas_call(
        _tc_node_body,
        out_shape=(shp((N, H), _F32), shp((N, 2 * H), _F32),
                   shp((1, H), _F32), shp((1, H), _F32)),
    )(nf, W_ff_node, bn, We_s, We_d, We_g, be_, Wv2, bv_, bg)

    BE_BLK = 8000
    n_eblk = E // BE_BLK
    eproj = pl.pallas_call(
        _tc_edge_body,
        grid=(n_eblk,),
        in_specs=[
            pl.BlockSpec((BE_BLK, 16), lambda i: (i, 0)),
            pl.BlockSpec((16, H), lambda i: (0, 0)),
            pl.BlockSpec((1, H), lambda i: (0, 0)),
            pl.BlockSpec((H, H), lambda i: (0, 0)),
            pl.BlockSpec((1, H), lambda i: (0, 0)),
        ],
        out_specs=pl.BlockSpec((BE_BLK, H), lambda i: (i, 0)),
        out_shape=shp((E, H), _F32),
    )(ef, W_ff_edge, be1, We_e, ebias)

    sc_edge_kernel, sc_readout_kernel = _sc_kernels()
    zero_rows = jnp.zeros((N_PAD, 2 * H), _F32)
    aggp = sc_edge_kernel(T, eproj, src, dst, bond4atom, zero_rows)

    s2, t = pl.pallas_call(
        _tc2_body,
        out_shape=(shp((N, 2), _F32), shp((CP, 1), _F32)),
        compiler_params=pltpu.CompilerParams(
            vmem_limit_bytes=100 * 1024 * 1024),
    )(aggp[0, :N, :H], aggp[1, :N, :H], node0, Wv0, Wv1, vbias, W2, ed, Woe,
      bo_)

    out_flat = sc_readout_kernel(s2.reshape(2 * N), t.reshape(CP), i0, i1)
    return out_flat.reshape(C, P)


# speedup vs baseline: 3.3268x; 1.3068x over previous
import functools

import jax
import jax.numpy as jnp
from jax import lax
from jax.experimental import pallas as pl
from jax.experimental.pallas import tpu as pltpu
from jax.experimental.pallas import tpu_sc as plsc

N = 10000
E = 320000
H = 64
C = 1024
P = 16
CP = C * P

NUM_CORES = 2
NUM_SUBCORES = 16
NW = NUM_CORES * NUM_SUBCORES
EB = 128
NBATCH = E // EB
NB_BASE = NBATCH // NW
NB_REM = NBATCH % NW
N_PAD = 10112
ROWS_PER_TILE = N_PAD // NUM_SUBCORES

_F32 = jnp.float32


def _tc_node_body(nf, wn, bn, wes, wed, weg, be_, wv2, bv_, bg,
                  o_n0, o_ab, o_eb, o_vb):
    n0 = jnp.maximum(
        jnp.dot(nf[...], wn[...], preferred_element_type=_F32, precision=lax.Precision.HIGHEST) + bn[...], 0.0)
    o_n0[...] = n0
    a = jnp.dot(n0, wes[...], preferred_element_type=_F32, precision=lax.Precision.HIGHEST)
    b = jnp.dot(n0, wed[...], preferred_element_type=_F32, precision=lax.Precision.HIGHEST)
    o_ab[...] = jnp.concatenate([a, b], axis=1)
    g = jnp.maximum(bg[...], 0.0)
    o_eb[...] = jnp.dot(g, weg[...], preferred_element_type=_F32, precision=lax.Precision.HIGHEST) + be_[...]
    o_vb[...] = jnp.dot(g, wv2[...], preferred_element_type=_F32, precision=lax.Precision.HIGHEST) + bv_[...]


def _dot3(x, w):
    bf16 = jnp.bfloat16
    xh = x.astype(bf16)
    xl = (x - xh.astype(_F32)).astype(bf16)
    wh = w.astype(bf16)
    wl = (w - wh.astype(_F32)).astype(bf16)
    return (jnp.dot(xh, wh, preferred_element_type=_F32)
            + jnp.dot(xh, wl, preferred_element_type=_F32)
            + jnp.dot(xl, wh, preferred_element_type=_F32))


def _tc_edge_body(ef, we1, be1, wee, eb, o):
    e0 = jnp.maximum(_dot3(ef[...], we1[...]) + be1[...], 0.0)
    o[...] = _dot3(e0, wee[...]) + eb[...]


def _tc2_body(ag0, ag1, n0r, wv0, wv1, vb, w2, ed, woe, bo_, o_s2, o_t):
    aggsum = ag0[...] + ag1[...]
    nn = jnp.maximum(
        jnp.dot(aggsum, wv0[...], preferred_element_type=_F32, precision=lax.Precision.HIGHEST)
        + jnp.dot(n0r[...], wv1[...], preferred_element_type=_F32, precision=lax.Precision.HIGHEST)
        + vb[...], 0.0)
    nf = n0r[...] + nn
    o_s2[...] = jnp.dot(nf, w2[...], preferred_element_type=_F32, precision=lax.Precision.HIGHEST)
    o_t[...] = jnp.dot(ed[...], woe[...], preferred_element_type=_F32, precision=lax.Precision.HIGHEST) + bo_[...]


_CP_PER_W = CP // NW


@functools.cache
def _sc_kernels():
    mesh = plsc.VectorSubcoreMesh(
        core_axis_name="c", subcore_axis_name="s",
        num_cores=NUM_CORES, num_subcores=NUM_SUBCORES)

    @functools.partial(
        pl.kernel,
        mesh=mesh,
        out_type=jax.ShapeDtypeStruct((NUM_CORES, N_PAD, 2 * H), _F32),
        scratch_types=[
            pltpu.VMEM((1, EB), jnp.int32),
            pltpu.VMEM((1, EB), jnp.int32),
            pltpu.VMEM((1, EB), jnp.int32),
            pltpu.VMEM((EB, 2 * H), _F32),
            pltpu.VMEM((EB, 2 * H), _F32),
            pltpu.VMEM((EB, H), _F32),
            pltpu.VMEM_SHARED((N_PAD, 2 * H), _F32),
            pltpu.SemaphoreType.DMA,
            pltpu.SemaphoreType.DMA,
        ],
    )
    def sc_edge_kernel(t_hbm, ep_hbm, src_hbm, dst_hbm, seg_hbm,
                       zero_hbm, out_hbm, idxs, idxd, idxg, ra, rb, re,
                       agg_sh, sem1, sem2):
        cid = lax.axis_index("c")
        sid = lax.axis_index("s")
        wid = sid * NUM_CORES + cid
        r0 = sid * ROWS_PER_TILE
        pltpu.sync_copy(zero_hbm.at[pl.ds(r0, ROWS_PER_TILE)],
                        agg_sh.at[pl.ds(r0, ROWS_PER_TILE)])
        plsc.subcore_barrier()

        nb = NB_BASE + jnp.where(wid < NB_REM, 1, 0)

        def batch_body(j, carry):
            base = (wid + NW * j) * EB
            pltpu.sync_copy(src_hbm.at[pl.ds(base, EB)], idxs.at[0])
            pltpu.sync_copy(dst_hbm.at[pl.ds(base, EB)], idxd.at[0])
            pltpu.sync_copy(seg_hbm.at[pl.ds(base, EB)], idxg.at[0])
            cp_a = pltpu.async_copy(t_hbm.at[idxs.at[0]], ra, sem1)
            cp_b = pltpu.async_copy(t_hbm.at[idxd.at[0]], rb, sem2)
            pltpu.sync_copy(ep_hbm.at[pl.ds(base, EB)], re)
            cp_a.wait()
            cp_b.wait()

            def row_body(r, c2):
                for q in range(H // 16):
                    sl = pl.ds(q * 16, 16)
                    slb = pl.ds(H + q * 16, 16)
                    ra[r, sl] = jnp.maximum(
                        ra[r, sl] + rb[r, slb] + re[r, sl], 0.0)
                return c2

            lax.fori_loop(0, EB, row_body, 0)
            pltpu.sync_copy(ra, agg_sh.at[idxg.at[0]], add=True)
            return carry

        lax.fori_loop(0, nb, batch_body, 0)
        plsc.subcore_barrier()
        pltpu.sync_copy(agg_sh.at[pl.ds(r0, ROWS_PER_TILE)],
                        out_hbm.at[cid, pl.ds(r0, ROWS_PER_TILE)])

    @functools.partial(
        pl.kernel,
        mesh=mesh,
        out_type=jax.ShapeDtypeStruct((CP,), _F32),
        compiler_params=pltpu.CompilerParams(needs_layout_passes=False),
        scratch_types=[
            pltpu.VMEM((2 * N,), _F32),
            pltpu.VMEM((_CP_PER_W,), jnp.int32),
            pltpu.VMEM((_CP_PER_W,), jnp.int32),
            pltpu.VMEM((_CP_PER_W,), _F32),
            pltpu.VMEM((_CP_PER_W,), _F32),
        ],
    )
    def sc_readout_kernel(s2_hbm, t_hbm, i0_hbm, i1_hbm, out_hbm,
                          s2v, i0v, i1v, tv, ov):
        cid = lax.axis_index("c")
        sid = lax.axis_index("s")
        wid = sid * NUM_CORES + cid
        base = wid * _CP_PER_W
        pltpu.sync_copy(s2_hbm, s2v)
        pltpu.sync_copy(i0_hbm.at[pl.ds(base, _CP_PER_W)], i0v)
        pltpu.sync_copy(i1_hbm.at[pl.ds(base, _CP_PER_W)], i1v)
        pltpu.sync_copy(t_hbm.at[pl.ds(base, _CP_PER_W)], tv)

        def body(i, c):
            sl = pl.ds(i * 16, 16)
            a0 = i0v[sl] * 2
            a1 = i1v[sl] * 2 + 1
            v0 = plsc.load_gather(s2v, [a0])
            v1 = plsc.load_gather(s2v, [a1])
            ov[sl] = v0 + v1 + tv[sl]
            return c

        lax.fori_loop(0, _CP_PER_W // 16, body, 0)
        pltpu.sync_copy(ov, out_hbm.at[pl.ds(base, _CP_PER_W)])

    return sc_edge_kernel, sc_readout_kernel


def kernel(node_ftr, edge_ftr, edge_data, W_ff_node, b_ff_node, W_ff_edge,
           b_ff_edge, W_ff_gbl, b_ff_gbl, We, be, Wv, bv, Wu, bu, Wo, bo,
           atom4bond, bond4atom, combinations):
    nf = node_ftr[0]
    ef = edge_ftr[0]
    src = atom4bond[0]
    dst = atom4bond[1]
    i0 = combinations[..., 0].reshape(-1)
    i1 = combinations[..., 1].reshape(-1)
    ed = edge_data.reshape(CP, 4)

    We_s, We_d, We_e, We_g = We[0:H], We[H:2 * H], We[2 * H:3 * H], We[3 * H:]
    Wv0, Wv1, Wv2 = Wv[0:H], Wv[H:2 * H], Wv[2 * H:]
    W2 = jnp.stack([Wo[0:H, 0], Wo[H:2 * H, 0]], axis=1)
    Woe = Wo[2 * H:, :]

    bn = b_ff_node[None, :]
    be1 = b_ff_edge[None, :]
    be_ = be[None, :]
    bv_ = bv[None, :]
    bg = b_ff_gbl[None, :]
    bo_ = bo[None, :]

    shp = jax.ShapeDtypeStruct
    node0, T, ebias, vbias = pl.pallas_call(
        _tc_node_body,
        out_shape=(shp((N, H), _F32), shp((N, 2 * H), _F32),
                   shp((1, H), _F32), shp((1, H), _F32)),
    )(nf, W_ff_node, bn, We_s, We_d, We_g, be_, Wv2, bv_, bg)

    BE_BLK = 8000
    n_eblk = E // BE_BLK
    eproj = pl.pallas_call(
        _tc_edge_body,
        grid=(n_eblk,),
        in_specs=[
            pl.BlockSpec((BE_BLK, 16), lambda i: (i, 0)),
            pl.BlockSpec((16, H), lambda i: (0, 0)),
            pl.BlockSpec((1, H), lambda i: (0, 0)),
            pl.BlockSpec((H, H), lambda i: (0, 0)),
            pl.BlockSpec((1, H), lambda i: (0, 0)),
        ],
        out_specs=pl.BlockSpec((BE_BLK, H), lambda i: (i, 0)),
        out_shape=shp((E, H), _F32),
    )(ef, W_ff_edge, be1, We_e, ebias)

    sc_edge_kernel, sc_readout_kernel = _sc_kernels()
    zero_rows = jnp.zeros((N_PAD, 2 * H), _F32)
    aggp = sc_edge_kernel(T, eproj, src, dst, bond4atom, zero_rows)

    s2, t = pl.pallas_call(
        _tc2_body,
        out_shape=(shp((N, 2), _F32), shp((CP, 1), _F32)),
        compiler_params=pltpu.CompilerParams(
            vmem_limit_bytes=100 * 1024 * 1024),
    )(aggp[0, :N, :H], aggp[1, :N, :H], node0, Wv0, Wv1, vbias, W2, ed, Woe,
      bo_)

    out_flat = sc_readout_kernel(s2.reshape(2 * N), t.reshape(CP), i0, i1)
    return out_flat.reshape(C, P)


# speedup vs baseline: 3.4572x; 1.0392x over previous
import functools

import jax
import jax.numpy as jnp
from jax import lax
from jax.experimental import pallas as pl
from jax.experimental.pallas import tpu as pltpu
from jax.experimental.pallas import tpu_sc as plsc

N = 10000
E = 320000
H = 64
C = 1024
P = 16
CP = C * P

NUM_CORES = 2
NUM_SUBCORES = 16
NW = NUM_CORES * NUM_SUBCORES
EB = 128
K_CH = 2
EH = E // K_CH
N_PAD = 10112
ROWS_PER_TILE = N_PAD // NUM_SUBCORES

_F32 = jnp.float32


def _tc_node_body(nf, wn, bn, wes, wed, weg, be_, wv2, bv_, bg,
                  o_n0, o_ab, o_eb, o_vb):
    n0 = jnp.maximum(
        jnp.dot(nf[...], wn[...], preferred_element_type=_F32, precision=lax.Precision.HIGHEST) + bn[...], 0.0)
    o_n0[...] = n0
    a = jnp.dot(n0, wes[...], preferred_element_type=_F32, precision=lax.Precision.HIGHEST)
    b = jnp.dot(n0, wed[...], preferred_element_type=_F32, precision=lax.Precision.HIGHEST)
    o_ab[...] = jnp.concatenate([a, b], axis=1)
    g = jnp.maximum(bg[...], 0.0)
    o_eb[...] = jnp.dot(g, weg[...], preferred_element_type=_F32, precision=lax.Precision.HIGHEST) + be_[...]
    o_vb[...] = jnp.dot(g, wv2[...], preferred_element_type=_F32, precision=lax.Precision.HIGHEST) + bv_[...]


def _dot3(x, w):
    bf16 = jnp.bfloat16
    xh = x.astype(bf16)
    xl = (x - xh.astype(_F32)).astype(bf16)
    wh = w.astype(bf16)
    wl = (w - wh.astype(_F32)).astype(bf16)
    return (jnp.dot(xh, wh, preferred_element_type=_F32)
            + jnp.dot(xh, wl, preferred_element_type=_F32)
            + jnp.dot(xl, wh, preferred_element_type=_F32))


def _tc_edge_body(ef, we1, be1, wee, eb, o):
    e0 = jnp.maximum(_dot3(ef[...], we1[...]) + be1[...], 0.0)
    o[...] = _dot3(e0, wee[...]) + eb[...]


def _tc2_body(ag0, ag1, ag2, ag3, n0r, wv0, wv1, vb, w2, o_s2):
    aggsum = ag0[...] + ag1[...] + ag2[...] + ag3[...]
    nn = jnp.maximum(
        jnp.dot(aggsum, wv0[...], preferred_element_type=_F32, precision=lax.Precision.HIGHEST)
        + jnp.dot(n0r[...], wv1[...], preferred_element_type=_F32, precision=lax.Precision.HIGHEST)
        + vb[...], 0.0)
    nf = n0r[...] + nn
    o_s2[...] = jnp.dot(nf, w2[...], preferred_element_type=_F32, precision=lax.Precision.HIGHEST)


def _tc_t_body(edr, wblk, bo_, o_t):
    o_t[...] = jnp.dot(edr[...], wblk[...], preferred_element_type=_F32,
                       precision=lax.Precision.HIGHEST) + bo_[...]


_CP_PER_W = CP // NW


@functools.cache
def _sc_edge(nbatch):
    nb_base = nbatch // NW
    nb_rem = nbatch % NW
    mesh = plsc.VectorSubcoreMesh(
        core_axis_name="c", subcore_axis_name="s",
        num_cores=NUM_CORES, num_subcores=NUM_SUBCORES)

    @functools.partial(
        pl.kernel,
        mesh=mesh,
        out_type=jax.ShapeDtypeStruct((NUM_CORES, N_PAD, 2 * H), _F32),
        scratch_types=[
            pltpu.VMEM((1, EB), jnp.int32),
            pltpu.VMEM((1, EB), jnp.int32),
            pltpu.VMEM((1, EB), jnp.int32),
            pltpu.VMEM((EB, 2 * H), _F32),
            pltpu.VMEM((EB, 2 * H), _F32),
            pltpu.VMEM((EB, H), _F32),
            pltpu.VMEM_SHARED((N_PAD, 2 * H), _F32),
            pltpu.SemaphoreType.DMA,
            pltpu.SemaphoreType.DMA,
        ],
    )
    def sc_edge_kernel(t_hbm, ep_hbm, src_hbm, dst_hbm, seg_hbm,
                       zero_hbm, out_hbm, idxs, idxd, idxg, ra, rb, re,
                       agg_sh, sem1, sem2):
        cid = lax.axis_index("c")
        sid = lax.axis_index("s")
        wid = sid * NUM_CORES + cid
        r0 = sid * ROWS_PER_TILE
        pltpu.sync_copy(zero_hbm.at[pl.ds(r0, ROWS_PER_TILE)],
                        agg_sh.at[pl.ds(r0, ROWS_PER_TILE)])
        plsc.subcore_barrier()

        nb = nb_base + jnp.where(wid < nb_rem, 1, 0)

        def batch_body(j, carry):
            base = (wid + NW * j) * EB
            pltpu.sync_copy(src_hbm.at[pl.ds(base, EB)], idxs.at[0])
            pltpu.sync_copy(dst_hbm.at[pl.ds(base, EB)], idxd.at[0])
            pltpu.sync_copy(seg_hbm.at[pl.ds(base, EB)], idxg.at[0])
            cp_a = pltpu.async_copy(t_hbm.at[idxs.at[0]], ra, sem1)
            cp_b = pltpu.async_copy(t_hbm.at[idxd.at[0]], rb, sem2)
            pltpu.sync_copy(ep_hbm.at[pl.ds(base, EB)], re)
            cp_a.wait()
            cp_b.wait()

            def row_body(r, c2):
                for q in range(H // 16):
                    sl = pl.ds(q * 16, 16)
                    slb = pl.ds(H + q * 16, 16)
                    ra[r, sl] = jnp.maximum(
                        ra[r, sl] + rb[r, slb] + re[r, sl], 0.0)
                return c2

            lax.fori_loop(0, EB, row_body, 0)
            pltpu.sync_copy(ra, agg_sh.at[idxg.at[0]], add=True)
            return carry

        lax.fori_loop(0, nb, batch_body, 0)
        plsc.subcore_barrier()
        pltpu.sync_copy(agg_sh.at[pl.ds(r0, ROWS_PER_TILE)],
                        out_hbm.at[cid, pl.ds(r0, ROWS_PER_TILE)])

    return sc_edge_kernel


@functools.cache
def _sc_readout():
    mesh = plsc.VectorSubcoreMesh(
        core_axis_name="c", subcore_axis_name="s",
        num_cores=NUM_CORES, num_subcores=NUM_SUBCORES)

    @functools.partial(
        pl.kernel,
        mesh=mesh,
        out_type=jax.ShapeDtypeStruct((CP,), _F32),
        compiler_params=pltpu.CompilerParams(needs_layout_passes=False),
        scratch_types=[
            pltpu.VMEM((2 * N,), _F32),
            pltpu.VMEM((_CP_PER_W,), jnp.int32),
            pltpu.VMEM((_CP_PER_W,), jnp.int32),
            pltpu.VMEM((_CP_PER_W,), _F32),
            pltpu.VMEM((_CP_PER_W,), _F32),
        ],
    )
    def sc_readout_kernel(s2_hbm, t_hbm, i0_hbm, i1_hbm, out_hbm,
                          s2v, i0v, i1v, tv, ov):
        cid = lax.axis_index("c")
        sid = lax.axis_index("s")
        wid = sid * NUM_CORES + cid
        base = wid * _CP_PER_W
        pltpu.sync_copy(s2_hbm, s2v)
        pltpu.sync_copy(i0_hbm.at[pl.ds(base, _CP_PER_W)], i0v)
        pltpu.sync_copy(i1_hbm.at[pl.ds(base, _CP_PER_W)], i1v)
        pltpu.sync_copy(t_hbm.at[pl.ds(base, _CP_PER_W)], tv)

        def body(i, c):
            sl = pl.ds(i * 16, 16)
            a0 = i0v[sl] * 2
            a1 = i1v[sl] * 2 + 1
            v0 = plsc.load_gather(s2v, [a0])
            v1 = plsc.load_gather(s2v, [a1])
            ov[sl] = v0 + v1 + tv[sl]
            return c

        lax.fori_loop(0, _CP_PER_W // 16, body, 0)
        pltpu.sync_copy(ov, out_hbm.at[pl.ds(base, _CP_PER_W)])

    return sc_readout_kernel


def kernel(node_ftr, edge_ftr, edge_data, W_ff_node, b_ff_node, W_ff_edge,
           b_ff_edge, W_ff_gbl, b_ff_gbl, We, be, Wv, bv, Wu, bu, Wo, bo,
           atom4bond, bond4atom, combinations):
    nf = node_ftr[0]
    ef = edge_ftr[0]
    src = atom4bond[0]
    dst = atom4bond[1]
    i0 = combinations[..., 0].reshape(-1)
    i1 = combinations[..., 1].reshape(-1)
    ed = edge_data.reshape(CP, 4)

    We_s, We_d, We_e, We_g = We[0:H], We[H:2 * H], We[2 * H:3 * H], We[3 * H:]
    Wv0, Wv1, Wv2 = Wv[0:H], Wv[H:2 * H], Wv[2 * H:]
    W2 = jnp.stack([Wo[0:H, 0], Wo[H:2 * H, 0]], axis=1)
    Woe = Wo[2 * H:, :]

    bn = b_ff_node[None, :]
    be1 = b_ff_edge[None, :]
    be_ = be[None, :]
    bv_ = bv[None, :]
    bg = b_ff_gbl[None, :]
    bo_ = bo[None, :]

    shp = jax.ShapeDtypeStruct
    node0, T, ebias, vbias = pl.pallas_call(
        _tc_node_body,
        out_shape=(shp((N, H), _F32), shp((N, 2 * H), _F32),
                   shp((1, H), _F32), shp((1, H), _F32)),
    )(nf, W_ff_node, bn, We_s, We_d, We_g, be_, Wv2, bv_, bg)

    BE_BLK = 8000
    n_eblk = EH // BE_BLK
    sc_edge_kernel = _sc_edge(EH // EB)
    zero_rows = jnp.zeros((N_PAD, 2 * H), _F32)

    aggps = []
    for h in range(K_CH):
        sl = slice(h * EH, (h + 1) * EH)
        eproj_h = pl.pallas_call(
            _tc_edge_body,
            grid=(n_eblk,),
            in_specs=[
                pl.BlockSpec((BE_BLK, 16), lambda i: (i, 0)),
                pl.BlockSpec((16, H), lambda i: (0, 0)),
                pl.BlockSpec((1, H), lambda i: (0, 0)),
                pl.BlockSpec((H, H), lambda i: (0, 0)),
                pl.BlockSpec((1, H), lambda i: (0, 0)),
            ],
            out_specs=pl.BlockSpec((BE_BLK, H), lambda i: (i, 0)),
            out_shape=shp((EH, H), _F32),
        )(ef[sl], W_ff_edge, be1, We_e, ebias)
        aggps.append(
            sc_edge_kernel(T, eproj_h, src[sl], dst[sl], bond4atom[sl],
                           zero_rows))

    NB_N = 5
    NBLK = N // NB_N
    agg_spec = pl.BlockSpec((NBLK, H), lambda i: (i, 0))
    w_spec = pl.BlockSpec((H, H), lambda i: (0, 0))
    s2 = pl.pallas_call(
        _tc2_body,
        grid=(NB_N,),
        in_specs=[agg_spec, agg_spec, agg_spec, agg_spec, agg_spec,
                  w_spec, w_spec,
                  pl.BlockSpec((1, H), lambda i: (0, 0)),
                  pl.BlockSpec((H, 2), lambda i: (0, 0))],
        out_specs=pl.BlockSpec((NBLK, 2), lambda i: (i, 0)),
        out_shape=shp((N, 2), _F32),
    )(aggps[0][0, :N, :H], aggps[0][1, :N, :H],
      aggps[1][0, :N, :H], aggps[1][1, :N, :H],
      node0, Wv0, Wv1, vbias, W2)

    ed_rs = ed.reshape(CP // 32, 128)
    wblk = jnp.zeros((128, 32), _F32)
    cidx = jnp.arange(32)
    ridx = 4 * cidx[:, None] + jnp.arange(4)[None, :]
    wblk = wblk.at[ridx.reshape(-1), jnp.repeat(cidx, 4)].set(
        jnp.tile(Woe[:, 0], 32))
    t = pl.pallas_call(
        _tc_t_body,
        out_shape=shp((CP // 32, 32), _F32),
    )(ed_rs, wblk, bo[None, :])

    out_flat = _sc_readout()(s2.reshape(2 * N), t.reshape(CP), i0, i1)
    return out_flat.reshape(C, P)


# speedup vs baseline: 3.7574x; 1.0868x over previous
import functools

import jax
import jax.numpy as jnp
from jax import lax
from jax.experimental import pallas as pl
from jax.experimental.pallas import tpu as pltpu
from jax.experimental.pallas import tpu_sc as plsc

N = 10000
E = 320000
H = 64
C = 1024
P = 16
CP = C * P

NUM_CORES = 2
NUM_SUBCORES = 16
NW = NUM_CORES * NUM_SUBCORES
EB = 128
K_CH = 4
EH = E // K_CH
N_PAD = 10112
ROWS_PER_TILE = N_PAD // NUM_SUBCORES

_F32 = jnp.float32


def _tc_node_body(nf, wn, bn, wes, wed, weg, be_, wv2, bv_, bg,
                  o_n0, o_ab, o_eb, o_vb):
    n0 = jnp.maximum(
        jnp.dot(nf[...], wn[...], preferred_element_type=_F32, precision=lax.Precision.HIGHEST) + bn[...], 0.0)
    o_n0[...] = n0
    a = jnp.dot(n0, wes[...], preferred_element_type=_F32, precision=lax.Precision.HIGHEST)
    b = jnp.dot(n0, wed[...], preferred_element_type=_F32, precision=lax.Precision.HIGHEST)
    o_ab[...] = jnp.concatenate([a, b], axis=1)
    g = jnp.maximum(bg[...], 0.0)
    o_eb[...] = jnp.dot(g, weg[...], preferred_element_type=_F32, precision=lax.Precision.HIGHEST) + be_[...]
    o_vb[...] = jnp.dot(g, wv2[...], preferred_element_type=_F32, precision=lax.Precision.HIGHEST) + bv_[...]


def _dot3(x, w):
    bf16 = jnp.bfloat16
    xh = x.astype(bf16)
    xl = (x - xh.astype(_F32)).astype(bf16)
    wh = w.astype(bf16)
    wl = (w - wh.astype(_F32)).astype(bf16)
    return (jnp.dot(xh, wh, preferred_element_type=_F32)
            + jnp.dot(xh, wl, preferred_element_type=_F32)
            + jnp.dot(xl, wh, preferred_element_type=_F32))


def _tc_edge_body(ef, we1, be1, wee, eb, o):
    e0 = jnp.maximum(_dot3(ef[...], we1[...]) + be1[...], 0.0)
    o[...] = _dot3(e0, wee[...]) + eb[...]


def _tc2_body(ag0, ag1, ag2, ag3, ag4, ag5, ag6, ag7, n0r, wv0, wv1, vb, w2,
              o_s2):
    aggsum = (ag0[...] + ag1[...] + ag2[...] + ag3[...]
              + ag4[...] + ag5[...] + ag6[...] + ag7[...])
    nn = jnp.maximum(
        jnp.dot(aggsum, wv0[...], preferred_element_type=_F32, precision=lax.Precision.HIGHEST)
        + jnp.dot(n0r[...], wv1[...], preferred_element_type=_F32, precision=lax.Precision.HIGHEST)
        + vb[...], 0.0)
    nf = n0r[...] + nn
    o_s2[...] = jnp.dot(nf, w2[...], preferred_element_type=_F32, precision=lax.Precision.HIGHEST)


def _tc_t_body(edr, wblk, bo_, o_t):
    o_t[...] = jnp.dot(edr[...], wblk[...], preferred_element_type=_F32,
                       precision=lax.Precision.HIGHEST) + bo_[...]


_CP_PER_W = CP // NW


@functools.cache
def _sc_edge(nbatch, chunk_base):
    nb_base = nbatch // NW
    nb_rem = nbatch % NW
    mesh = plsc.VectorSubcoreMesh(
        core_axis_name="c", subcore_axis_name="s",
        num_cores=NUM_CORES, num_subcores=NUM_SUBCORES)

    @functools.partial(
        pl.kernel,
        mesh=mesh,
        out_type=jax.ShapeDtypeStruct((NUM_CORES, N_PAD, 2 * H), _F32),
        scratch_types=[
            pltpu.VMEM((1, EB), jnp.int32),
            pltpu.VMEM((1, EB), jnp.int32),
            pltpu.VMEM((1, EB), jnp.int32),
            pltpu.VMEM((EB, 2 * H), _F32),
            pltpu.VMEM((EB, 2 * H), _F32),
            pltpu.VMEM((EB, H), _F32),
            pltpu.VMEM_SHARED((N_PAD, 2 * H), _F32),
            pltpu.SemaphoreType.DMA,
            pltpu.SemaphoreType.DMA,
        ],
    )
    def sc_edge_kernel(t_hbm, ep_hbm, src_hbm, dst_hbm, seg_hbm,
                       zero_hbm, out_hbm, idxs, idxd, idxg, ra, rb, re,
                       agg_sh, sem1, sem2):
        cid = lax.axis_index("c")
        sid = lax.axis_index("s")
        wid = sid * NUM_CORES + cid
        r0 = sid * ROWS_PER_TILE
        pltpu.sync_copy(zero_hbm.at[pl.ds(r0, ROWS_PER_TILE)],
                        agg_sh.at[pl.ds(r0, ROWS_PER_TILE)])
        plsc.subcore_barrier()

        nb = nb_base + jnp.where(wid < nb_rem, 1, 0)

        def batch_body(j, carry):
            base = (wid + NW * j) * EB
            gbase = chunk_base + base
            pltpu.sync_copy(src_hbm.at[pl.ds(gbase, EB)], idxs.at[0])
            pltpu.sync_copy(dst_hbm.at[pl.ds(gbase, EB)], idxd.at[0])
            pltpu.sync_copy(seg_hbm.at[pl.ds(gbase, EB)], idxg.at[0])
            cp_a = pltpu.async_copy(t_hbm.at[idxs.at[0]], ra, sem1)
            cp_b = pltpu.async_copy(t_hbm.at[idxd.at[0]], rb, sem2)
            pltpu.sync_copy(ep_hbm.at[pl.ds(base, EB)], re)
            cp_a.wait()
            cp_b.wait()

            def row_body(r, c2):
                for q in range(H // 16):
                    sl = pl.ds(q * 16, 16)
                    slb = pl.ds(H + q * 16, 16)
                    ra[r, sl] = jnp.maximum(
                        ra[r, sl] + rb[r, slb] + re[r, sl], 0.0)
                return c2

            lax.fori_loop(0, EB, row_body, 0)
            pltpu.sync_copy(ra, agg_sh.at[idxg.at[0]], add=True)
            return carry

        lax.fori_loop(0, nb, batch_body, 0)
        plsc.subcore_barrier()
        pltpu.sync_copy(agg_sh.at[pl.ds(r0, ROWS_PER_TILE)],
                        out_hbm.at[cid, pl.ds(r0, ROWS_PER_TILE)])

    return sc_edge_kernel


@functools.cache
def _sc_readout():
    mesh = plsc.VectorSubcoreMesh(
        core_axis_name="c", subcore_axis_name="s",
        num_cores=NUM_CORES, num_subcores=NUM_SUBCORES)

    @functools.partial(
        pl.kernel,
        mesh=mesh,
        out_type=jax.ShapeDtypeStruct((CP,), _F32),
        compiler_params=pltpu.CompilerParams(needs_layout_passes=False),
        scratch_types=[
            pltpu.VMEM((2 * N,), _F32),
            pltpu.VMEM((_CP_PER_W,), jnp.int32),
            pltpu.VMEM((_CP_PER_W,), jnp.int32),
            pltpu.VMEM((_CP_PER_W,), _F32),
            pltpu.VMEM((_CP_PER_W,), _F32),
        ],
    )
    def sc_readout_kernel(s2_hbm, t_hbm, i0_hbm, i1_hbm, out_hbm,
                          s2v, i0v, i1v, tv, ov):
        cid = lax.axis_index("c")
        sid = lax.axis_index("s")
        wid = sid * NUM_CORES + cid
        base = wid * _CP_PER_W
        pltpu.sync_copy(s2_hbm, s2v)
        pltpu.sync_copy(i0_hbm.at[pl.ds(base, _CP_PER_W)], i0v)
        pltpu.sync_copy(i1_hbm.at[pl.ds(base, _CP_PER_W)], i1v)
        pltpu.sync_copy(t_hbm.at[pl.ds(base, _CP_PER_W)], tv)

        def body(i, c):
            sl = pl.ds(i * 16, 16)
            a0 = i0v[sl] * 2
            a1 = i1v[sl] * 2 + 1
            v0 = plsc.load_gather(s2v, [a0])
            v1 = plsc.load_gather(s2v, [a1])
            ov[sl] = v0 + v1 + tv[sl]
            return c

        lax.fori_loop(0, _CP_PER_W // 16, body, 0)
        pltpu.sync_copy(ov, out_hbm.at[pl.ds(base, _CP_PER_W)])

    return sc_readout_kernel


def kernel(node_ftr, edge_ftr, edge_data, W_ff_node, b_ff_node, W_ff_edge,
           b_ff_edge, W_ff_gbl, b_ff_gbl, We, be, Wv, bv, Wu, bu, Wo, bo,
           atom4bond, bond4atom, combinations):
    nf = node_ftr[0]
    ef = edge_ftr[0]
    src = atom4bond[0]
    dst = atom4bond[1]
    i0 = combinations[..., 0].reshape(-1)
    i1 = combinations[..., 1].reshape(-1)
    ed = edge_data.reshape(CP, 4)

    We_s, We_d, We_e, We_g = We[0:H], We[H:2 * H], We[2 * H:3 * H], We[3 * H:]
    Wv0, Wv1, Wv2 = Wv[0:H], Wv[H:2 * H], Wv[2 * H:]
    W2 = jnp.stack([Wo[0:H, 0], Wo[H:2 * H, 0]], axis=1)
    Woe = Wo[2 * H:, :]

    bn = b_ff_node[None, :]
    be1 = b_ff_edge[None, :]
    be_ = be[None, :]
    bv_ = bv[None, :]
    bg = b_ff_gbl[None, :]
    bo_ = bo[None, :]

    shp = jax.ShapeDtypeStruct
    node0, T, ebias, vbias = pl.pallas_call(
        _tc_node_body,
        out_shape=(shp((N, H), _F32), shp((N, 2 * H), _F32),
                   shp((1, H), _F32), shp((1, H), _F32)),
    )(nf, W_ff_node, bn, We_s, We_d, We_g, be_, Wv2, bv_, bg)

    BE_BLK = 8000
    n_eblk = EH // BE_BLK
    zero_rows = jnp.zeros((N_PAD, 2 * H), _F32)

    aggps = []
    for h in range(K_CH):
        eproj_h = pl.pallas_call(
            _tc_edge_body,
            grid=(n_eblk,),
            in_specs=[
                pl.BlockSpec((BE_BLK, 16),
                             functools.partial(lambda h_, i: (i + h_ * n_eblk, 0), h)),
                pl.BlockSpec((16, H), lambda i: (0, 0)),
                pl.BlockSpec((1, H), lambda i: (0, 0)),
                pl.BlockSpec((H, H), lambda i: (0, 0)),
                pl.BlockSpec((1, H), lambda i: (0, 0)),
            ],
            out_specs=pl.BlockSpec((BE_BLK, H), lambda i: (i, 0)),
            out_shape=shp((EH, H), _F32),
        )(ef, W_ff_edge, be1, We_e, ebias)
        aggps.append(
            _sc_edge(EH // EB, h * EH)(T, eproj_h, src, dst, bond4atom,
                                       zero_rows))

    NB_N = 5
    NBLK = N // NB_N
    agg_spec = pl.BlockSpec((NBLK, H), lambda i: (i, 0))
    w_spec = pl.BlockSpec((H, H), lambda i: (0, 0))
    agg_args = [aggps[h][c, :N, :H] for h in range(K_CH) for c in range(2)]
    s2 = pl.pallas_call(
        _tc2_body,
        grid=(NB_N,),
        in_specs=[agg_spec] * (2 * K_CH + 1) + [
                  w_spec, w_spec,
                  pl.BlockSpec((1, H), lambda i: (0, 0)),
                  pl.BlockSpec((H, 2), lambda i: (0, 0))],
        out_specs=pl.BlockSpec((NBLK, 2), lambda i: (i, 0)),
        out_shape=shp((N, 2), _F32),
    )(*agg_args, node0, Wv0, Wv1, vbias, W2)

    ed_rs = ed.reshape(CP // 32, 128)
    wblk = jnp.zeros((128, 32), _F32)
    cidx = jnp.arange(32)
    ridx = 4 * cidx[:, None] + jnp.arange(4)[None, :]
    wblk = wblk.at[ridx.reshape(-1), jnp.repeat(cidx, 4)].set(
        jnp.tile(Woe[:, 0], 32))
    t = pl.pallas_call(
        _tc_t_body,
        out_shape=shp((CP // 32, 32), _F32),
    )(ed_rs, wblk, bo[None, :])

    out_flat = _sc_readout()(s2.reshape(2 * N), t.reshape(CP), i0, i1)
    return out_flat.reshape(C, P)
